# Initial kernel scaffold; baseline (speedup 1.0000x reference)
import jax
import jax.numpy as jnp
from jax.experimental import pallas as pl


def kernel(x, emb, src_ids):
    raise NotImplementedError("write your pallas kernel here")


# speedup vs baseline: 1.5369x; 1.5369x over previous
import jax
import jax.numpy as jnp
from jax.experimental import pallas as pl
from jax.experimental.pallas import tpu as pltpu

_BP = 256


def _body(ids_ref, emb_ref, x_ref, o_ref):
    ids = ids_ref[...]
    emb = emb_ref[...]
    n_sources, d = emb.shape
    enc = jnp.zeros((ids.shape[0], d), jnp.float32)
    for s in range(n_sources):
        enc = jnp.where(ids == s, emb[s].reshape(1, d), enc)
    o_ref[...] = x_ref[...] + enc[:, None, :]


def kernel(x, emb, src_ids):
    total, batch, d = x.shape
    grid = total // _BP
    ids2 = src_ids.reshape(total, 1)
    return pl.pallas_call(
        _body,
        grid=(grid,),
        in_specs=[
            pl.BlockSpec((_BP, 1), lambda i: (i, 0)),
            pl.BlockSpec(emb.shape, lambda i: (0, 0)),
            pl.BlockSpec((_BP, batch, d), lambda i: (i, 0, 0)),
        ],
        out_specs=pl.BlockSpec((_BP, batch, d), lambda i: (i, 0, 0)),
        out_shape=jax.ShapeDtypeStruct(x.shape, x.dtype),
    )(ids2, emb, x)
